# Initial kernel scaffold; baseline (speedup 1.0000x reference)
import jax
import jax.numpy as jnp
from jax.experimental import pallas as pl


def kernel(x, edge_index, W, att_src, att_dst, bias):
    raise NotImplementedError("write your pallas kernel here")


# speedup vs baseline: 57.9866x; 57.9866x over previous
import functools

import jax
import jax.numpy as jnp
from jax import lax
from jax.experimental import pallas as pl
from jax.experimental.pallas import tpu as pltpu
from jax.experimental.pallas import tpu_sc as plsc

E = 160000
D = 128


_BLK = 640
_NBLK = E // _BLK


def _tc_body(x_ref, w_ref, as_ref, ad_ref, b_ref, h_ref, s_ref, d_ref):
    h = jnp.dot(x_ref[...], w_ref[...], preferred_element_type=jnp.float32)
    s_ref[...] = jnp.sum(h * as_ref[...], axis=1).reshape(1, 1, _BLK)
    d_ref[...] = jnp.sum(h * ad_ref[...], axis=1).reshape(1, 1, _BLK)
    h_ref[...] = h + b_ref[...]


_tc_call = pl.pallas_call(
    _tc_body,
    grid=(_NBLK,),
    in_specs=[
        pl.BlockSpec((_BLK, D), lambda i: (i, 0)),
        pl.BlockSpec((D, D), lambda i: (0, 0)),
        pl.BlockSpec((1, D), lambda i: (0, 0)),
        pl.BlockSpec((1, D), lambda i: (0, 0)),
        pl.BlockSpec((1, D), lambda i: (0, 0)),
    ],
    out_specs=[
        pl.BlockSpec((_BLK, D), lambda i: (i, 0)),
        pl.BlockSpec((1, 1, _BLK), lambda i: (i, 0, 0)),
        pl.BlockSpec((1, 1, _BLK), lambda i: (i, 0, 0)),
    ],
    out_shape=[
        jax.ShapeDtypeStruct((E, D), jnp.float32),
        jax.ShapeDtypeStruct((_NBLK, 1, _BLK), jnp.float32),
        jax.ShapeDtypeStruct((_NBLK, 1, _BLK), jnp.float32),
    ],
)


_NC = 2
_NS = 16
_NW = _NC * _NS
_L = 16
_C = 128
_NCHUNK = E // _C
_SLOTS = -(-_NCHUNK // _NW)


def _sc_body(ea_hbm, eb_hbm, h_hbm, s_hbm, d_hbm, out_hbm,
             idx_a, idx_b, ra, rb, rs, ro, sa, sb, ss, dd, sem):
    wid = lax.axis_index("s") * _NC + lax.axis_index("c")

    def chunk_body(slot, carry):
        chunk = slot * _NW + wid

        @pl.when(chunk < _NCHUNK)
        def _():
            base = chunk * _C
            pltpu.sync_copy(ea_hbm.at[pl.ds(base, _C)], idx_a)
            pltpu.sync_copy(eb_hbm.at[pl.ds(base, _C)], idx_b)
            cps = [
                pltpu.async_copy(h_hbm.at[idx_a], ra, sem),
                pltpu.async_copy(h_hbm.at[idx_b], rb, sem),
                pltpu.async_copy(s_hbm.at[idx_a], sa, sem),
                pltpu.async_copy(s_hbm.at[idx_b], sb, sem),
                pltpu.async_copy(h_hbm.at[pl.ds(base, _C)], rs, sem),
                pltpu.async_copy(s_hbm.at[pl.ds(base, _C)], ss, sem),
                pltpu.async_copy(d_hbm.at[pl.ds(base, _C)], dd, sem),
            ]
            for cp in cps:
                cp.wait()

            def group_body(g, gcarry):
                gs = pl.ds(g * _L, _L)
                va = sa[gs]
                vb = sb[gs]
                vs = ss[gs]
                vd = dd[gs]
                e1 = va + vd
                e2 = vb + vd
                e3 = vs + vd
                e1 = jnp.where(e1 > 0, e1, 0.2 * e1)
                e2 = jnp.where(e2 > 0, e2, 0.2 * e2)
                e3 = jnp.where(e3 > 0, e3, 0.2 * e3)
                m = jnp.maximum(e1, jnp.maximum(e2, e3))
                x1 = jnp.exp(e1 - m)
                x2 = jnp.exp(e2 - m)
                x3 = jnp.exp(e3 - m)
                inv = 1.0 / (x1 + x2 + x3 + 1e-16)
                a1v = x1 * inv
                a2v = x2 * inv
                a3v = x3 * inv

                def edge_body(j, ecarry):
                    r = g * _L + j
                    jj = jnp.zeros((_L,), jnp.int32) + j
                    b1 = a1v.at[jj].get(mode="promise_in_bounds")
                    b2 = a2v.at[jj].get(mode="promise_in_bounds")
                    b3 = a3v.at[jj].get(mode="promise_in_bounds")
                    for c in range(D // _L):
                        cs = pl.ds(c * _L, _L)
                        ro[r, cs] = (b1 * ra[r, cs] + b2 * rb[r, cs]
                                     + b3 * rs[r, cs])
                    return ecarry

                lax.fori_loop(0, _L, edge_body, 0, unroll=False)
                return gcarry

            lax.fori_loop(0, _C // _L, group_body, 0, unroll=False)
            pltpu.sync_copy(ro, out_hbm.at[pl.ds(base, _C)])

        return carry

    lax.fori_loop(0, _SLOTS, chunk_body, 0, unroll=False)


_sc_call = functools.partial(
    pl.kernel,
    mesh=plsc.VectorSubcoreMesh(core_axis_name="c", subcore_axis_name="s"),
    out_type=jax.ShapeDtypeStruct((E, D), jnp.float32),
    scratch_types=[
        pltpu.VMEM((_C,), jnp.int32),
        pltpu.VMEM((_C,), jnp.int32),
        pltpu.VMEM((_C, D), jnp.float32),
        pltpu.VMEM((_C, D), jnp.float32),
        pltpu.VMEM((_C, D), jnp.float32),
        pltpu.VMEM((_C, D), jnp.float32),
        pltpu.VMEM((_C,), jnp.float32),
        pltpu.VMEM((_C,), jnp.float32),
        pltpu.VMEM((_C,), jnp.float32),
        pltpu.VMEM((_C,), jnp.float32),
        pltpu.SemaphoreType.DMA,
    ],
)(_sc_body)


def kernel(x, edge_index, W, att_src, att_dst, bias):
    h, s2, d2 = _tc_call(x, W, att_src.reshape(1, D), att_dst.reshape(1, D),
                         bias.reshape(1, D))
    s = s2.reshape(E)
    d = d2.reshape(E)
    return _sc_call(edge_index[0], edge_index[1], h, s, d)


# speedup vs baseline: 74.7535x; 1.2892x over previous
import functools

import jax
import jax.numpy as jnp
from jax import lax
from jax.experimental import pallas as pl
from jax.experimental.pallas import tpu as pltpu
from jax.experimental.pallas import tpu_sc as plsc

E = 160000
D = 128


_BLK = 640
_NBLK = E // _BLK


def _tc_body(x_ref, w_ref, as_ref, ad_ref, b_ref, h_ref, s_ref, d_ref):
    h = jnp.dot(x_ref[...], w_ref[...], preferred_element_type=jnp.float32)
    s_ref[...] = jnp.sum(h * as_ref[...], axis=1).reshape(1, 1, _BLK)
    d_ref[...] = jnp.sum(h * ad_ref[...], axis=1).reshape(1, 1, _BLK)
    h_ref[...] = h + b_ref[...]


_tc_call = pl.pallas_call(
    _tc_body,
    grid=(_NBLK,),
    in_specs=[
        pl.BlockSpec((_BLK, D), lambda i: (i, 0)),
        pl.BlockSpec((D, D), lambda i: (0, 0)),
        pl.BlockSpec((1, D), lambda i: (0, 0)),
        pl.BlockSpec((1, D), lambda i: (0, 0)),
        pl.BlockSpec((1, D), lambda i: (0, 0)),
    ],
    out_specs=[
        pl.BlockSpec((_BLK, D), lambda i: (i, 0)),
        pl.BlockSpec((1, 1, _BLK), lambda i: (i, 0, 0)),
        pl.BlockSpec((1, 1, _BLK), lambda i: (i, 0, 0)),
    ],
    out_shape=[
        jax.ShapeDtypeStruct((E, D), jnp.float32),
        jax.ShapeDtypeStruct((_NBLK, 1, _BLK), jnp.float32),
        jax.ShapeDtypeStruct((_NBLK, 1, _BLK), jnp.float32),
    ],
)


_NC = 2
_NS = 16
_NW = _NC * _NS
_L = 16
_C = 128
_NCHUNK = E // _C
_BASE = _NCHUNK // _NW
_REM = _NCHUNK % _NW
_SLOTS = _BASE + 1


def _sc_body(ea_hbm, eb_hbm, h_hbm, s_hbm, d_hbm, out_hbm,
             ia, ib, ssl, ddl,
             ra0, rb0, rs0, sa0, sb0,
             ra1, rb1, rs1, sa1, sb1,
             gsem0, gsem1, wsem0, wsem1):
    wid = lax.axis_index("s") * _NC + lax.axis_index("c")
    nch = _BASE + (wid < _REM).astype(jnp.int32)
    cbase = wid * _BASE + jnp.minimum(wid, _REM)

    ebase = cbase * _C
    _B = _BASE * _C

    pltpu.sync_copy(ea_hbm.at[pl.ds(ebase, _B)], ia.at[pl.ds(0, _B)])
    pltpu.sync_copy(eb_hbm.at[pl.ds(ebase, _B)], ib.at[pl.ds(0, _B)])
    pltpu.sync_copy(s_hbm.at[pl.ds(ebase, _B)], ssl.at[pl.ds(0, _B)])
    pltpu.sync_copy(d_hbm.at[pl.ds(ebase, _B)], ddl.at[pl.ds(0, _B)])

    @pl.when(nch > _BASE)
    def _():
        pltpu.sync_copy(ea_hbm.at[pl.ds(ebase + _B, _C)], ia.at[pl.ds(_B, _C)])
        pltpu.sync_copy(eb_hbm.at[pl.ds(ebase + _B, _C)], ib.at[pl.ds(_B, _C)])
        pltpu.sync_copy(s_hbm.at[pl.ds(ebase + _B, _C)], ssl.at[pl.ds(_B, _C)])
        pltpu.sync_copy(d_hbm.at[pl.ds(ebase + _B, _C)], ddl.at[pl.ds(_B, _C)])

    bufs = ((ra0, rb0, rs0, sa0, sb0, gsem0, wsem0),
            (ra1, rb1, rs1, sa1, sb1, gsem1, wsem1))

    def issue(t, buf):
        ra, rb, rs, sa, sb, gsem, wsem = buf

        @pl.when(t < nch)
        def _():
            base = (cbase + t) * _C

            @pl.when(t >= 2)
            def _():
                pltpu.make_async_copy(
                    ra, out_hbm.at[pl.ds(0, _C)], wsem).wait()

            pltpu.async_copy(h_hbm.at[ia.at[pl.ds(t * _C, _C)]], ra, gsem)
            pltpu.async_copy(h_hbm.at[ib.at[pl.ds(t * _C, _C)]], rb, gsem)
            pltpu.async_copy(s_hbm.at[ia.at[pl.ds(t * _C, _C)]], sa, gsem)
            pltpu.async_copy(s_hbm.at[ib.at[pl.ds(t * _C, _C)]], sb, gsem)
            pltpu.async_copy(h_hbm.at[pl.ds(base, _C)], rs, gsem)

    def process(t, buf):
        ra, rb, rs, sa, sb, gsem, wsem = buf

        @pl.when(t < nch)
        def _():
            base = (cbase + t) * _C
            pltpu.make_async_copy(h_hbm.at[ia.at[pl.ds(t * _C, _C)]], ra, gsem).wait()
            pltpu.make_async_copy(h_hbm.at[ib.at[pl.ds(t * _C, _C)]], rb, gsem).wait()
            pltpu.make_async_copy(s_hbm.at[ia.at[pl.ds(t * _C, _C)]], sa, gsem).wait()
            pltpu.make_async_copy(s_hbm.at[ib.at[pl.ds(t * _C, _C)]], sb, gsem).wait()
            pltpu.make_async_copy(h_hbm.at[pl.ds(base, _C)], rs, gsem).wait()

            def group_body(g, gcarry):
                gs = pl.ds(g * _L, _L)
                va = sa[gs]
                vb = sb[gs]
                ts = pl.ds(t * _C + g * _L, _L)
                vs = ssl[ts]
                vd = ddl[ts]
                e1 = va + vd
                e2 = vb + vd
                e3 = vs + vd
                e1 = jnp.where(e1 > 0, e1, 0.2 * e1)
                e2 = jnp.where(e2 > 0, e2, 0.2 * e2)
                e3 = jnp.where(e3 > 0, e3, 0.2 * e3)
                m = jnp.maximum(e1, jnp.maximum(e2, e3))
                x1 = jnp.exp(e1 - m)
                x2 = jnp.exp(e2 - m)
                x3 = jnp.exp(e3 - m)
                inv = 1.0 / (x1 + x2 + x3 + 1e-16)
                a1v = x1 * inv
                a2v = x2 * inv
                a3v = x3 * inv

                def edge_body(j, ecarry):
                    r = g * _L + j
                    jj = jnp.zeros((_L,), jnp.int32) + j
                    b1 = a1v.at[jj].get(mode="promise_in_bounds")
                    b2 = a2v.at[jj].get(mode="promise_in_bounds")
                    b3 = a3v.at[jj].get(mode="promise_in_bounds")
                    for c in range(D // _L):
                        cs = pl.ds(c * _L, _L)
                        ra[r, cs] = (b1 * ra[r, cs] + b2 * rb[r, cs]
                                     + b3 * rs[r, cs])
                    return ecarry

                lax.fori_loop(0, _L, edge_body, 0, unroll=False)
                return gcarry

            lax.fori_loop(0, _C // _L, group_body, 0, unroll=False)
            pltpu.async_copy(ra, out_hbm.at[pl.ds(base, _C)], wsem)

    issue(0, bufs[0])

    def pair_body(k, carry):
        t0 = 2 * k
        issue(t0 + 1, bufs[1])
        process(t0, bufs[0])
        issue(t0 + 2, bufs[0])
        process(t0 + 1, bufs[1])
        return carry

    lax.fori_loop(0, _SLOTS // 2, pair_body, 0, unroll=False)

    for b in range(2):
        ra = bufs[b][0]
        wsem = bufs[b][6]
        pltpu.make_async_copy(ra, out_hbm.at[pl.ds(0, _C)], wsem).wait()


_sc_call = functools.partial(
    pl.kernel,
    mesh=plsc.VectorSubcoreMesh(core_axis_name="c", subcore_axis_name="s"),
    out_type=jax.ShapeDtypeStruct((E, D), jnp.float32),
    scratch_types=(
        [pltpu.VMEM((_SLOTS * _C,), jnp.int32)] * 2
        + [pltpu.VMEM((_SLOTS * _C,), jnp.float32)] * 2
        + [pltpu.VMEM((_C, D), jnp.float32),
           pltpu.VMEM((_C, D), jnp.float32),
           pltpu.VMEM((_C, D), jnp.float32),
           pltpu.VMEM((_C,), jnp.float32),
           pltpu.VMEM((_C,), jnp.float32)] * 2
        + [pltpu.SemaphoreType.DMA] * 4
    ),
)(_sc_body)


def kernel(x, edge_index, W, att_src, att_dst, bias):
    h, s2, d2 = _tc_call(x, W, att_src.reshape(1, D), att_dst.reshape(1, D),
                         bias.reshape(1, D))
    s = s2.reshape(E)
    d = d2.reshape(E)
    return _sc_call(edge_index[0], edge_index[1], h, s, d)


# speedup vs baseline: 80.9697x; 1.0832x over previous
import functools

import jax
import jax.numpy as jnp
from jax import lax
from jax.experimental import pallas as pl
from jax.experimental.pallas import tpu as pltpu
from jax.experimental.pallas import tpu_sc as plsc

E = 160000
D = 128


_BLK = 640
_NBLK = E // _BLK


def _tc_body(x_ref, w_ref, a2_ref, b_ref, h_ref, sd_ref):
    h = jnp.dot(x_ref[...], w_ref[...], preferred_element_type=jnp.float32)
    sd = lax.dot_general(a2_ref[...], h, (((1,), (1,)), ((), ())),
                         preferred_element_type=jnp.float32)
    sd_ref[...] = sd.reshape(1, 2, _BLK)
    h_ref[...] = h + b_ref[...]


_tc_call = pl.pallas_call(
    _tc_body,
    grid=(_NBLK,),
    in_specs=[
        pl.BlockSpec((_BLK, D), lambda i: (i, 0)),
        pl.BlockSpec((D, D), lambda i: (0, 0)),
        pl.BlockSpec((2, D), lambda i: (0, 0)),
        pl.BlockSpec((1, D), lambda i: (0, 0)),
    ],
    out_specs=[
        pl.BlockSpec((_BLK, D), lambda i: (i, 0)),
        pl.BlockSpec((1, 2, _BLK), lambda i: (i, 0, 0)),
    ],
    out_shape=[
        jax.ShapeDtypeStruct((E, D), jnp.float32),
        jax.ShapeDtypeStruct((_NBLK, 2, _BLK), jnp.float32),
    ],
)


_NC = 2
_NS = 16
_NW = _NC * _NS
_L = 16
_C = 128
_NCHUNK = E // _C
_BASE = _NCHUNK // _NW
_REM = _NCHUNK % _NW
_SLOTS = _BASE + 1


def _sc_body(ea_hbm, eb_hbm, h_hbm, s_hbm, d_hbm, out_hbm,
             ia, ib, ssl, ddl,
             ra0, rb0, rs0, sa0, sb0,
             ra1, rb1, rs1, sa1, sb1,
             gsem0, gsem1, wsem0, wsem1):
    wid = lax.axis_index("s") * _NC + lax.axis_index("c")
    nch = _BASE + (wid < _REM).astype(jnp.int32)
    cbase = wid * _BASE + jnp.minimum(wid, _REM)

    ebase = cbase * _C
    _B = _BASE * _C

    pltpu.sync_copy(ea_hbm.at[pl.ds(ebase, _B)], ia.at[pl.ds(0, _B)])
    pltpu.sync_copy(eb_hbm.at[pl.ds(ebase, _B)], ib.at[pl.ds(0, _B)])
    pltpu.sync_copy(s_hbm.at[pl.ds(ebase, _B)], ssl.at[pl.ds(0, _B)])
    pltpu.sync_copy(d_hbm.at[pl.ds(ebase, _B)], ddl.at[pl.ds(0, _B)])

    @pl.when(nch > _BASE)
    def _():
        pltpu.sync_copy(ea_hbm.at[pl.ds(ebase + _B, _C)], ia.at[pl.ds(_B, _C)])
        pltpu.sync_copy(eb_hbm.at[pl.ds(ebase + _B, _C)], ib.at[pl.ds(_B, _C)])
        pltpu.sync_copy(s_hbm.at[pl.ds(ebase + _B, _C)], ssl.at[pl.ds(_B, _C)])
        pltpu.sync_copy(d_hbm.at[pl.ds(ebase + _B, _C)], ddl.at[pl.ds(_B, _C)])

    bufs = ((ra0, rb0, rs0, sa0, sb0, gsem0, wsem0),
            (ra1, rb1, rs1, sa1, sb1, gsem1, wsem1))

    def issue(t, buf):
        ra, rb, rs, sa, sb, gsem, wsem = buf

        @pl.when(t < nch)
        def _():
            base = (cbase + t) * _C

            @pl.when(t >= 2)
            def _():
                pltpu.make_async_copy(
                    ra, out_hbm.at[pl.ds(0, _C)], wsem).wait()

            pltpu.async_copy(h_hbm.at[ia.at[pl.ds(t * _C, _C)]], ra, gsem)
            pltpu.async_copy(h_hbm.at[ib.at[pl.ds(t * _C, _C)]], rb, gsem)
            pltpu.async_copy(s_hbm.at[ia.at[pl.ds(t * _C, _C)]], sa, gsem)
            pltpu.async_copy(s_hbm.at[ib.at[pl.ds(t * _C, _C)]], sb, gsem)
            pltpu.async_copy(h_hbm.at[pl.ds(base, _C)], rs, gsem)

    def process(t, buf):
        ra, rb, rs, sa, sb, gsem, wsem = buf

        @pl.when(t < nch)
        def _():
            base = (cbase + t) * _C
            pltpu.make_async_copy(h_hbm.at[ia.at[pl.ds(t * _C, _C)]], ra, gsem).wait()
            pltpu.make_async_copy(h_hbm.at[ib.at[pl.ds(t * _C, _C)]], rb, gsem).wait()
            pltpu.make_async_copy(s_hbm.at[ia.at[pl.ds(t * _C, _C)]], sa, gsem).wait()
            pltpu.make_async_copy(s_hbm.at[ib.at[pl.ds(t * _C, _C)]], sb, gsem).wait()
            pltpu.make_async_copy(h_hbm.at[pl.ds(base, _C)], rs, gsem).wait()

            def group_body(g, gcarry):
                gs = pl.ds(g * _L, _L)
                va = sa[gs]
                vb = sb[gs]
                ts = pl.ds(t * _C + g * _L, _L)
                vs = ssl[ts]
                vd = ddl[ts]
                e1 = va + vd
                e2 = vb + vd
                e3 = vs + vd
                e1 = jnp.where(e1 > 0, e1, 0.2 * e1)
                e2 = jnp.where(e2 > 0, e2, 0.2 * e2)
                e3 = jnp.where(e3 > 0, e3, 0.2 * e3)
                m = jnp.maximum(e1, jnp.maximum(e2, e3))
                x1 = jnp.exp(e1 - m)
                x2 = jnp.exp(e2 - m)
                x3 = jnp.exp(e3 - m)
                inv = 1.0 / (x1 + x2 + x3 + 1e-16)
                a1v = x1 * inv
                a2v = x2 * inv
                a3v = x3 * inv

                def edge_body(j, ecarry):
                    r = g * _L + j
                    jj = jnp.zeros((_L,), jnp.int32) + j
                    b1 = a1v.at[jj].get(mode="promise_in_bounds")
                    b2 = a2v.at[jj].get(mode="promise_in_bounds")
                    b3 = a3v.at[jj].get(mode="promise_in_bounds")
                    for c in range(D // _L):
                        cs = pl.ds(c * _L, _L)
                        ra[r, cs] = (b1 * ra[r, cs] + b2 * rb[r, cs]
                                     + b3 * rs[r, cs])
                    return ecarry

                lax.fori_loop(0, _L, edge_body, 0, unroll=False)
                return gcarry

            lax.fori_loop(0, _C // _L, group_body, 0, unroll=False)
            pltpu.async_copy(ra, out_hbm.at[pl.ds(base, _C)], wsem)

    issue(0, bufs[0])

    def pair_body(k, carry):
        t0 = 2 * k
        issue(t0 + 1, bufs[1])
        process(t0, bufs[0])
        issue(t0 + 2, bufs[0])
        process(t0 + 1, bufs[1])
        return carry

    lax.fori_loop(0, _SLOTS // 2, pair_body, 0, unroll=False)

    for b in range(2):
        ra = bufs[b][0]
        wsem = bufs[b][6]
        pltpu.make_async_copy(ra, out_hbm.at[pl.ds(0, _C)], wsem).wait()


_sc_call = functools.partial(
    pl.kernel,
    mesh=plsc.VectorSubcoreMesh(core_axis_name="c", subcore_axis_name="s"),
    out_type=jax.ShapeDtypeStruct((E, D), jnp.float32),
    scratch_types=(
        [pltpu.VMEM((_SLOTS * _C,), jnp.int32)] * 2
        + [pltpu.VMEM((_SLOTS * _C,), jnp.float32)] * 2
        + [pltpu.VMEM((_C, D), jnp.float32),
           pltpu.VMEM((_C, D), jnp.float32),
           pltpu.VMEM((_C, D), jnp.float32),
           pltpu.VMEM((_C,), jnp.float32),
           pltpu.VMEM((_C,), jnp.float32)] * 2
        + [pltpu.SemaphoreType.DMA] * 4
    ),
)(_sc_body)


def kernel(x, edge_index, W, att_src, att_dst, bias):
    a2 = jnp.stack([att_src, att_dst])
    h, sd = _tc_call(x, W, a2, bias.reshape(1, D))
    s = sd[:, 0, :].reshape(E)
    d = sd[:, 1, :].reshape(E)
    return _sc_call(edge_index[0], edge_index[1], h, s, d)


# speedup vs baseline: 116.4251x; 1.4379x over previous
import functools

import jax
import jax.numpy as jnp
from jax import lax
from jax.experimental import pallas as pl
from jax.experimental.pallas import tpu as pltpu
from jax.experimental.pallas import tpu_sc as plsc

E = 160000
D = 128


_BLK = 640
_NBLK = E // _BLK


def _tc_body(x_ref, w_ref, a2_ref, b_ref, h_ref, sd_ref):
    h = jnp.dot(x_ref[...], w_ref[...], preferred_element_type=jnp.float32)
    sd = lax.dot_general(a2_ref[...], h, (((1,), (1,)), ((), ())),
                         preferred_element_type=jnp.float32)
    sd_ref[...] = sd.reshape(1, 2, _BLK)
    h_ref[...] = h + b_ref[...]


_tc_call = pl.pallas_call(
    _tc_body,
    grid=(_NBLK,),
    in_specs=[
        pl.BlockSpec((_BLK, D), lambda i: (i, 0)),
        pl.BlockSpec((D, D), lambda i: (0, 0)),
        pl.BlockSpec((2, D), lambda i: (0, 0)),
        pl.BlockSpec((1, D), lambda i: (0, 0)),
    ],
    out_specs=[
        pl.BlockSpec((_BLK, D), lambda i: (i, 0)),
        pl.BlockSpec((1, 2, _BLK), lambda i: (i, 0, 0)),
    ],
    out_shape=[
        jax.ShapeDtypeStruct((E, D), jnp.float32),
        jax.ShapeDtypeStruct((_NBLK, 2, _BLK), jnp.float32),
    ],
)


_NC = 2
_NS = 16
_NW = _NC * _NS
_L = 16
_C = 128
_NCHUNK = E // _C
_BASE = _NCHUNK // _NW
_REM = _NCHUNK % _NW
_SLOTS = _BASE + 1


def _sc_body(ea_hbm, eb_hbm, h_hbm, s_hbm, d_hbm, out_hbm,
             ia, ib, ssl, ddl,
             ra0, rb0, rs0, sa0, sb0,
             ra1, rb1, rs1, sa1, sb1,
             gsem0, gsem1, wsem0, wsem1):
    wid = lax.axis_index("s") * _NC + lax.axis_index("c")
    nch = _BASE + (wid < _REM).astype(jnp.int32)
    cbase = wid * _BASE + jnp.minimum(wid, _REM)

    ebase = cbase * _C
    _B = _BASE * _C

    pltpu.sync_copy(ea_hbm.at[pl.ds(ebase, _B)], ia.at[pl.ds(0, _B)])
    pltpu.sync_copy(eb_hbm.at[pl.ds(ebase, _B)], ib.at[pl.ds(0, _B)])
    pltpu.sync_copy(s_hbm.at[pl.ds(ebase, _B)], ssl.at[pl.ds(0, _B)])
    pltpu.sync_copy(d_hbm.at[pl.ds(ebase, _B)], ddl.at[pl.ds(0, _B)])

    @pl.when(nch > _BASE)
    def _():
        pltpu.sync_copy(ea_hbm.at[pl.ds(ebase + _B, _C)], ia.at[pl.ds(_B, _C)])
        pltpu.sync_copy(eb_hbm.at[pl.ds(ebase + _B, _C)], ib.at[pl.ds(_B, _C)])
        pltpu.sync_copy(s_hbm.at[pl.ds(ebase + _B, _C)], ssl.at[pl.ds(_B, _C)])
        pltpu.sync_copy(d_hbm.at[pl.ds(ebase + _B, _C)], ddl.at[pl.ds(_B, _C)])

    bufs = ((ra0, rb0, rs0, sa0, sb0, gsem0, wsem0),
            (ra1, rb1, rs1, sa1, sb1, gsem1, wsem1))

    def issue(t, buf):
        ra, rb, rs, sa, sb, gsem, wsem = buf

        @pl.when(t < nch)
        def _():
            base = (cbase + t) * _C

            pltpu.async_copy(h_hbm.at[ib.at[pl.ds(t * _C, _C)]], rb, gsem)
            pltpu.async_copy(s_hbm.at[ia.at[pl.ds(t * _C, _C)]], sa, gsem)
            pltpu.async_copy(s_hbm.at[ib.at[pl.ds(t * _C, _C)]], sb, gsem)
            pltpu.async_copy(h_hbm.at[pl.ds(base, _C)], rs, gsem)

            @pl.when(t >= 2)
            def _():
                pltpu.make_async_copy(
                    ra, out_hbm.at[pl.ds(0, _C)], wsem).wait()

            pltpu.async_copy(h_hbm.at[ia.at[pl.ds(t * _C, _C)]], ra, gsem)

    def process(t, buf):
        ra, rb, rs, sa, sb, gsem, wsem = buf

        @pl.when(t < nch)
        def _():
            base = (cbase + t) * _C
            pltpu.make_async_copy(h_hbm.at[ia.at[pl.ds(t * _C, _C)]], ra, gsem).wait()
            pltpu.make_async_copy(h_hbm.at[ib.at[pl.ds(t * _C, _C)]], rb, gsem).wait()
            pltpu.make_async_copy(s_hbm.at[ia.at[pl.ds(t * _C, _C)]], sa, gsem).wait()
            pltpu.make_async_copy(s_hbm.at[ib.at[pl.ds(t * _C, _C)]], sb, gsem).wait()
            pltpu.make_async_copy(h_hbm.at[pl.ds(base, _C)], rs, gsem).wait()

            def group_body(g, gcarry):
                gs = pl.ds(g * _L, _L)
                va = sa[gs]
                vb = sb[gs]
                ts = pl.ds(t * _C + g * _L, _L)
                vs = ssl[ts]
                vd = ddl[ts]
                e1 = va + vd
                e2 = vb + vd
                e3 = vs + vd
                e1 = jnp.where(e1 > 0, e1, 0.2 * e1)
                e2 = jnp.where(e2 > 0, e2, 0.2 * e2)
                e3 = jnp.where(e3 > 0, e3, 0.2 * e3)
                m = jnp.maximum(e1, jnp.maximum(e2, e3))
                x1 = jnp.exp(e1 - m)
                x2 = jnp.exp(e2 - m)
                x3 = jnp.exp(e3 - m)
                inv = 1.0 / (x1 + x2 + x3 + 1e-16)
                a1v = x1 * inv
                a2v = x2 * inv
                a3v = x3 * inv

                @plsc.parallel_loop(0, _L, 1, unroll=2)
                def _edge(j):
                    r = g * _L + j
                    jj = jnp.zeros((_L,), jnp.int32) + j
                    b1 = a1v.at[jj].get(mode="promise_in_bounds")
                    b2 = a2v.at[jj].get(mode="promise_in_bounds")
                    b3 = a3v.at[jj].get(mode="promise_in_bounds")
                    for c in range(D // _L):
                        cs = pl.ds(c * _L, _L)
                        ra[r, cs] = (b1 * ra[r, cs] + b2 * rb[r, cs]
                                     + b3 * rs[r, cs])

                return gcarry

            lax.fori_loop(0, _C // _L, group_body, 0, unroll=False)
            pltpu.async_copy(ra, out_hbm.at[pl.ds(base, _C)], wsem)

    issue(0, bufs[0])

    def pair_body(k, carry):
        t0 = 2 * k
        issue(t0 + 1, bufs[1])
        process(t0, bufs[0])
        issue(t0 + 2, bufs[0])
        process(t0 + 1, bufs[1])
        return carry

    lax.fori_loop(0, _SLOTS // 2, pair_body, 0, unroll=False)

    for b in range(2):
        ra = bufs[b][0]
        wsem = bufs[b][6]
        pltpu.make_async_copy(ra, out_hbm.at[pl.ds(0, _C)], wsem).wait()


_sc_call = functools.partial(
    pl.kernel,
    mesh=plsc.VectorSubcoreMesh(core_axis_name="c", subcore_axis_name="s"),
    out_type=jax.ShapeDtypeStruct((E, D), jnp.float32),
    scratch_types=(
        [pltpu.VMEM((_SLOTS * _C,), jnp.int32)] * 2
        + [pltpu.VMEM((_SLOTS * _C,), jnp.float32)] * 2
        + [pltpu.VMEM((_C, D), jnp.float32),
           pltpu.VMEM((_C, D), jnp.float32),
           pltpu.VMEM((_C, D), jnp.float32),
           pltpu.VMEM((_C,), jnp.float32),
           pltpu.VMEM((_C,), jnp.float32)] * 2
        + [pltpu.SemaphoreType.DMA] * 4
    ),
)(_sc_body)


def kernel(x, edge_index, W, att_src, att_dst, bias):
    a2 = jnp.stack([att_src, att_dst])
    h, sd = _tc_call(x, W, a2, bias.reshape(1, D))
    s = sd[:, 0, :].reshape(E)
    d = sd[:, 1, :].reshape(E)
    return _sc_call(edge_index[0], edge_index[1], h, s, d)


# speedup vs baseline: 165.7684x; 1.4238x over previous
import functools

import jax
import jax.numpy as jnp
from jax import lax
from jax.experimental import pallas as pl
from jax.experimental.pallas import tpu as pltpu
from jax.experimental.pallas import tpu_sc as plsc

E = 160000
D = 128


_BLK = 3200
_NBLK = E // _BLK


def _tc_body(x_ref, w_ref, a2_ref, b_ref, h_ref, sd_ref):
    h = jnp.dot(x_ref[...], w_ref[...], preferred_element_type=jnp.float32)
    sd = lax.dot_general(a2_ref[...], h, (((1,), (1,)), ((), ())),
                         preferred_element_type=jnp.float32)
    sd_ref[...] = sd.reshape(1, 2, _BLK)
    h_ref[...] = h + b_ref[...]


_tc_call = pl.pallas_call(
    _tc_body,
    grid=(_NBLK,),
    in_specs=[
        pl.BlockSpec((_BLK, D), lambda i: (i, 0)),
        pl.BlockSpec((D, D), lambda i: (0, 0)),
        pl.BlockSpec((2, D), lambda i: (0, 0)),
        pl.BlockSpec((1, D), lambda i: (0, 0)),
    ],
    out_specs=[
        pl.BlockSpec((_BLK, D), lambda i: (i, 0)),
        pl.BlockSpec((1, 2, _BLK), lambda i: (i, 0, 0)),
    ],
    out_shape=[
        jax.ShapeDtypeStruct((E, D), jnp.float32),
        jax.ShapeDtypeStruct((_NBLK, 2, _BLK), jnp.float32),
    ],
)


_NC = 2
_NS = 16
_NW = _NC * _NS
_L = 16
_C = 128
_NCHUNK = E // _C
_BASE = _NCHUNK // _NW
_REM = _NCHUNK % _NW
_SLOTS = _BASE + 1


def _sc_body(ea_hbm, eb_hbm, h_hbm, s_hbm, d_hbm, out_hbm,
             ia, ib, ssl, ddl,
             ra0, rb0, rs0, sa0, sb0,
             ra1, rb1, rs1, sa1, sb1,
             gsem0, gsem1, wsem0, wsem1):
    wid = lax.axis_index("s") * _NC + lax.axis_index("c")
    nch = _BASE + (wid < _REM).astype(jnp.int32)
    cbase = wid * _BASE + jnp.minimum(wid, _REM)

    ebase = cbase * _C
    _B = _BASE * _C

    pltpu.sync_copy(ea_hbm.at[pl.ds(ebase, _B)], ia.at[pl.ds(0, _B)])
    pltpu.sync_copy(eb_hbm.at[pl.ds(ebase, _B)], ib.at[pl.ds(0, _B)])
    pltpu.sync_copy(s_hbm.at[pl.ds(ebase, _B)], ssl.at[pl.ds(0, _B)])
    pltpu.sync_copy(d_hbm.at[pl.ds(ebase, _B)], ddl.at[pl.ds(0, _B)])

    @pl.when(nch > _BASE)
    def _():
        pltpu.sync_copy(ea_hbm.at[pl.ds(ebase + _B, _C)], ia.at[pl.ds(_B, _C)])
        pltpu.sync_copy(eb_hbm.at[pl.ds(ebase + _B, _C)], ib.at[pl.ds(_B, _C)])
        pltpu.sync_copy(s_hbm.at[pl.ds(ebase + _B, _C)], ssl.at[pl.ds(_B, _C)])
        pltpu.sync_copy(d_hbm.at[pl.ds(ebase + _B, _C)], ddl.at[pl.ds(_B, _C)])

    bufs = ((ra0, rb0, rs0, sa0, sb0, gsem0, wsem0),
            (ra1, rb1, rs1, sa1, sb1, gsem1, wsem1))

    def issue(t, buf):
        ra, rb, rs, sa, sb, gsem, wsem = buf

        @pl.when(t < nch)
        def _():
            base = (cbase + t) * _C

            pltpu.async_copy(h_hbm.at[ib.at[pl.ds(t * _C, _C)]], rb, gsem)
            pltpu.async_copy(s_hbm.at[ia.at[pl.ds(t * _C, _C)]], sa, gsem)
            pltpu.async_copy(s_hbm.at[ib.at[pl.ds(t * _C, _C)]], sb, gsem)
            pltpu.async_copy(h_hbm.at[pl.ds(base, _C)], rs, gsem)

            @pl.when(t >= 2)
            def _():
                pltpu.make_async_copy(
                    ra, out_hbm.at[pl.ds(0, _C)], wsem).wait()

            pltpu.async_copy(h_hbm.at[ia.at[pl.ds(t * _C, _C)]], ra, gsem)

    def process(t, buf):
        ra, rb, rs, sa, sb, gsem, wsem = buf

        @pl.when(t < nch)
        def _():
            base = (cbase + t) * _C
            pltpu.make_async_copy(h_hbm.at[ia.at[pl.ds(t * _C, _C)]], ra, gsem).wait()
            pltpu.make_async_copy(h_hbm.at[ib.at[pl.ds(t * _C, _C)]], rb, gsem).wait()
            pltpu.make_async_copy(s_hbm.at[ia.at[pl.ds(t * _C, _C)]], sa, gsem).wait()
            pltpu.make_async_copy(s_hbm.at[ib.at[pl.ds(t * _C, _C)]], sb, gsem).wait()
            pltpu.make_async_copy(h_hbm.at[pl.ds(base, _C)], rs, gsem).wait()

            def group_body(g, gcarry):
                gs = pl.ds(g * _L, _L)
                va = sa[gs]
                vb = sb[gs]
                ts = pl.ds(t * _C + g * _L, _L)
                vs = ssl[ts]
                vd = ddl[ts]
                e1 = va + vd
                e2 = vb + vd
                e3 = vs + vd
                e1 = jnp.where(e1 > 0, e1, 0.2 * e1)
                e2 = jnp.where(e2 > 0, e2, 0.2 * e2)
                e3 = jnp.where(e3 > 0, e3, 0.2 * e3)
                m = jnp.maximum(e1, jnp.maximum(e2, e3))
                x1 = jnp.exp(e1 - m)
                x2 = jnp.exp(e2 - m)
                x3 = jnp.exp(e3 - m)
                inv = 1.0 / (x1 + x2 + x3 + 1e-16)
                a1v = x1 * inv
                a2v = x2 * inv
                a3v = x3 * inv

                @plsc.parallel_loop(0, _L, 1, unroll=2)
                def _edge(j):
                    r = g * _L + j
                    jj = jnp.zeros((_L,), jnp.int32) + j
                    b1 = a1v.at[jj].get(mode="promise_in_bounds")
                    b2 = a2v.at[jj].get(mode="promise_in_bounds")
                    b3 = a3v.at[jj].get(mode="promise_in_bounds")
                    for c in range(D // _L):
                        cs = pl.ds(c * _L, _L)
                        ra[r, cs] = (b1 * ra[r, cs] + b2 * rb[r, cs]
                                     + b3 * rs[r, cs])

                return gcarry

            lax.fori_loop(0, _C // _L, group_body, 0, unroll=False)
            pltpu.async_copy(ra, out_hbm.at[pl.ds(base, _C)], wsem)

    issue(0, bufs[0])

    def pair_body(k, carry):
        t0 = 2 * k
        issue(t0 + 1, bufs[1])
        process(t0, bufs[0])
        issue(t0 + 2, bufs[0])
        process(t0 + 1, bufs[1])
        return carry

    lax.fori_loop(0, _SLOTS // 2, pair_body, 0, unroll=False)

    for b in range(2):
        ra = bufs[b][0]
        wsem = bufs[b][6]
        pltpu.make_async_copy(ra, out_hbm.at[pl.ds(0, _C)], wsem).wait()


_sc_call = functools.partial(
    pl.kernel,
    mesh=plsc.VectorSubcoreMesh(core_axis_name="c", subcore_axis_name="s"),
    out_type=jax.ShapeDtypeStruct((E, D), jnp.float32),
    scratch_types=(
        [pltpu.VMEM((_SLOTS * _C,), jnp.int32)] * 2
        + [pltpu.VMEM((_SLOTS * _C,), jnp.float32)] * 2
        + [pltpu.VMEM((_C, D), jnp.float32),
           pltpu.VMEM((_C, D), jnp.float32),
           pltpu.VMEM((_C, D), jnp.float32),
           pltpu.VMEM((_C,), jnp.float32),
           pltpu.VMEM((_C,), jnp.float32)] * 2
        + [pltpu.SemaphoreType.DMA] * 4
    ),
)(_sc_body)


def kernel(x, edge_index, W, att_src, att_dst, bias):
    a2 = jnp.stack([att_src, att_dst])
    h, sd = _tc_call(x, W, a2, bias.reshape(1, D))
    s = sd[:, 0, :].reshape(E)
    d = sd[:, 1, :].reshape(E)
    return _sc_call(edge_index[0], edge_index[1], h, s, d)


# speedup vs baseline: 165.8113x; 1.0003x over previous
import functools

import jax
import jax.numpy as jnp
from jax import lax
from jax.experimental import pallas as pl
from jax.experimental.pallas import tpu as pltpu
from jax.experimental.pallas import tpu_sc as plsc

E = 160000
D = 128


_BLK = 3200
_NBLK = E // _BLK


def _tc_body(x_ref, w_ref, a2_ref, b_ref, h_ref, sd_ref):
    h = jnp.dot(x_ref[...], w_ref[...], preferred_element_type=jnp.float32)
    sd = lax.dot_general(a2_ref[...], h, (((1,), (1,)), ((), ())),
                         preferred_element_type=jnp.float32)
    sd_ref[...] = sd.reshape(1, 2, _BLK)
    h_ref[...] = h + b_ref[...]


_tc_call = pl.pallas_call(
    _tc_body,
    grid=(_NBLK,),
    in_specs=[
        pl.BlockSpec((_BLK, D), lambda i: (i, 0)),
        pl.BlockSpec((D, D), lambda i: (0, 0)),
        pl.BlockSpec((2, D), lambda i: (0, 0)),
        pl.BlockSpec((1, D), lambda i: (0, 0)),
    ],
    out_specs=[
        pl.BlockSpec((_BLK, D), lambda i: (i, 0)),
        pl.BlockSpec((1, 2, _BLK), lambda i: (i, 0, 0)),
    ],
    out_shape=[
        jax.ShapeDtypeStruct((E, D), jnp.float32),
        jax.ShapeDtypeStruct((_NBLK, 2, _BLK), jnp.float32),
    ],
)


_NC = 2
_NS = 16
_NW = _NC * _NS
_L = 16
_C = 128
_NCHUNK = E // _C
_BASE = _NCHUNK // _NW
_REM = _NCHUNK % _NW
_SLOTS = _BASE + 1


def _sc_body(ea_hbm, eb_hbm, h_hbm, s_hbm, d_hbm, out_hbm,
             ia, ib, ssl, ddl,
             ra0, rb0, rs0, sa0, sb0,
             ra1, rb1, rs1, sa1, sb1,
             gsem0, gsem1, wsem0, wsem1):
    wid = lax.axis_index("s") * _NC + lax.axis_index("c")
    nch = _BASE + (wid < _REM).astype(jnp.int32)
    cbase = wid * _BASE + jnp.minimum(wid, _REM)

    ebase = cbase * _C
    _B = _BASE * _C

    pltpu.sync_copy(ea_hbm.at[pl.ds(ebase, _B)], ia.at[pl.ds(0, _B)])
    pltpu.sync_copy(eb_hbm.at[pl.ds(ebase, _B)], ib.at[pl.ds(0, _B)])
    pltpu.sync_copy(s_hbm.at[pl.ds(ebase, _B)], ssl.at[pl.ds(0, _B)])
    pltpu.sync_copy(d_hbm.at[pl.ds(ebase, _B)], ddl.at[pl.ds(0, _B)])

    @pl.when(nch > _BASE)
    def _():
        pltpu.sync_copy(ea_hbm.at[pl.ds(ebase + _B, _C)], ia.at[pl.ds(_B, _C)])
        pltpu.sync_copy(eb_hbm.at[pl.ds(ebase + _B, _C)], ib.at[pl.ds(_B, _C)])
        pltpu.sync_copy(s_hbm.at[pl.ds(ebase + _B, _C)], ssl.at[pl.ds(_B, _C)])
        pltpu.sync_copy(d_hbm.at[pl.ds(ebase + _B, _C)], ddl.at[pl.ds(_B, _C)])

    bufs = ((ra0, rb0, rs0, sa0, sb0, gsem0, wsem0),
            (ra1, rb1, rs1, sa1, sb1, gsem1, wsem1))

    def issue(t, buf):
        ra, rb, rs, sa, sb, gsem, wsem = buf

        @pl.when(t < nch)
        def _():
            base = (cbase + t) * _C

            pltpu.async_copy(h_hbm.at[ib.at[pl.ds(t * _C, _C)]], rb, gsem)
            pltpu.async_copy(s_hbm.at[ia.at[pl.ds(t * _C, _C)]], sa, gsem)
            pltpu.async_copy(s_hbm.at[ib.at[pl.ds(t * _C, _C)]], sb, gsem)
            pltpu.async_copy(h_hbm.at[pl.ds(base, _C)], rs, gsem)

            @pl.when(t >= 2)
            def _():
                pltpu.make_async_copy(
                    ra, out_hbm.at[pl.ds(0, _C)], wsem).wait()

            pltpu.async_copy(h_hbm.at[ia.at[pl.ds(t * _C, _C)]], ra, gsem)

    def process(t, buf):
        ra, rb, rs, sa, sb, gsem, wsem = buf

        @pl.when(t < nch)
        def _():
            base = (cbase + t) * _C
            pltpu.make_async_copy(h_hbm.at[ia.at[pl.ds(t * _C, _C)]], ra, gsem).wait()
            pltpu.make_async_copy(h_hbm.at[ib.at[pl.ds(t * _C, _C)]], rb, gsem).wait()
            pltpu.make_async_copy(s_hbm.at[ia.at[pl.ds(t * _C, _C)]], sa, gsem).wait()
            pltpu.make_async_copy(s_hbm.at[ib.at[pl.ds(t * _C, _C)]], sb, gsem).wait()
            pltpu.make_async_copy(h_hbm.at[pl.ds(base, _C)], rs, gsem).wait()

            def group_body(g, gcarry):
                gs = pl.ds(g * _L, _L)
                va = sa[gs]
                vb = sb[gs]
                ts = pl.ds(t * _C + g * _L, _L)
                vs = ssl[ts]
                vd = ddl[ts]
                e1 = va + vd
                e2 = vb + vd
                e3 = vs + vd
                e1 = jnp.where(e1 > 0, e1, 0.2 * e1)
                e2 = jnp.where(e2 > 0, e2, 0.2 * e2)
                e3 = jnp.where(e3 > 0, e3, 0.2 * e3)
                m = jnp.maximum(e1, jnp.maximum(e2, e3))
                x1 = jnp.exp(e1 - m)
                x2 = jnp.exp(e2 - m)
                x3 = jnp.exp(e3 - m)
                inv = 1.0 / (x1 + x2 + x3 + 1e-16)
                a1v = x1 * inv
                a2v = x2 * inv
                a3v = x3 * inv

                @plsc.parallel_loop(0, _L, 1, unroll=4)
                def _edge(j):
                    r = g * _L + j
                    jj = jnp.zeros((_L,), jnp.int32) + j
                    b1 = a1v.at[jj].get(mode="promise_in_bounds")
                    b2 = a2v.at[jj].get(mode="promise_in_bounds")
                    b3 = a3v.at[jj].get(mode="promise_in_bounds")
                    for c in range(D // _L):
                        cs = pl.ds(c * _L, _L)
                        ra[r, cs] = (b1 * ra[r, cs] + b2 * rb[r, cs]
                                     + b3 * rs[r, cs])

                return gcarry

            lax.fori_loop(0, _C // _L, group_body, 0, unroll=False)
            pltpu.async_copy(ra, out_hbm.at[pl.ds(base, _C)], wsem)

    issue(0, bufs[0])

    def pair_body(k, carry):
        t0 = 2 * k
        issue(t0 + 1, bufs[1])
        process(t0, bufs[0])
        issue(t0 + 2, bufs[0])
        process(t0 + 1, bufs[1])
        return carry

    lax.fori_loop(0, _SLOTS // 2, pair_body, 0, unroll=False)

    for b in range(2):
        ra = bufs[b][0]
        wsem = bufs[b][6]
        pltpu.make_async_copy(ra, out_hbm.at[pl.ds(0, _C)], wsem).wait()


_sc_call = functools.partial(
    pl.kernel,
    mesh=plsc.VectorSubcoreMesh(core_axis_name="c", subcore_axis_name="s"),
    out_type=jax.ShapeDtypeStruct((E, D), jnp.float32),
    scratch_types=(
        [pltpu.VMEM((_SLOTS * _C,), jnp.int32)] * 2
        + [pltpu.VMEM((_SLOTS * _C,), jnp.float32)] * 2
        + [pltpu.VMEM((_C, D), jnp.float32),
           pltpu.VMEM((_C, D), jnp.float32),
           pltpu.VMEM((_C, D), jnp.float32),
           pltpu.VMEM((_C,), jnp.float32),
           pltpu.VMEM((_C,), jnp.float32)] * 2
        + [pltpu.SemaphoreType.DMA] * 4
    ),
)(_sc_body)


def kernel(x, edge_index, W, att_src, att_dst, bias):
    a2 = jnp.stack([att_src, att_dst])
    h, sd = _tc_call(x, W, a2, bias.reshape(1, D))
    s = sd[:, 0, :].reshape(E)
    d = sd[:, 1, :].reshape(E)
    return _sc_call(edge_index[0], edge_index[1], h, s, d)
